# baseline (speedup 1.0000x reference)
import functools

import jax
import jax.numpy as jnp
from jax import lax
from jax.experimental import pallas as pl
from jax.experimental.pallas import tpu as pltpu
from jax.experimental.pallas import tpu_sc as plsc

B = 16384
D = 64
TAG = 128
NW = 32
BPW = B // NW
CH = 128
NCH = BPW // CH


def _gather_body(uidx_hbm, aidx_hbm, user_hbm, artist_hbm, u_out, a_out,
                 uidx_v, aidx_v, urows_v, arows_v, sem):
    wid = lax.axis_index("s") * 2 + lax.axis_index("c")
    base = wid * BPW
    pltpu.sync_copy(uidx_hbm.at[wid], uidx_v)
    pltpu.sync_copy(aidx_hbm.at[wid], aidx_v)
    descs = []
    for j in range(NCH):
        descs.append(pltpu.async_copy(
            user_hbm.at[uidx_v.at[j]], urows_v.at[pl.ds(j * CH, CH)], sem))
        descs.append(pltpu.async_copy(
            artist_hbm.at[aidx_v.at[j]], arows_v.at[pl.ds(j * CH, CH)], sem))
    for dsc in descs:
        dsc.wait()
    pltpu.sync_copy(urows_v, u_out.at[pl.ds(base, BPW)])
    pltpu.sync_copy(arows_v, a_out.at[pl.ds(base, BPW)])


_gather = functools.partial(
    pl.kernel,
    mesh=plsc.VectorSubcoreMesh(core_axis_name="c", subcore_axis_name="s"),
    out_type=(jax.ShapeDtypeStruct((B, D), jnp.float32),
              jax.ShapeDtypeStruct((B, D), jnp.float32)),
    scratch_types=[
        pltpu.VMEM((NCH, CH), jnp.int32),
        pltpu.VMEM((NCH, CH), jnp.int32),
        pltpu.VMEM((BPW, D), jnp.float32),
        pltpu.VMEM((BPW, D), jnp.float32),
        pltpu.SemaphoreType.DMA,
    ],
    compiler_params=pltpu.CompilerParams(use_tc_tiling_on_sc=False),
)(_gather_body)


BB = 2048


def _mlp_body(u_ref, a_ref, t_ref, wtag_ref, w1u_ref, w1a_ref, w1t_ref,
              b1_ref, w2_ref, b2_ref, w3_ref, b3_ref, out_ref):
    f32 = jnp.float32
    t = jnp.maximum(
        jnp.dot(t_ref[...], wtag_ref[...], preferred_element_type=f32), 0.0)
    h = jnp.dot(u_ref[...], w1u_ref[...], preferred_element_type=f32)
    h = h + jnp.dot(a_ref[...], w1a_ref[...], preferred_element_type=f32)
    h = h + jnp.dot(t, w1t_ref[...], preferred_element_type=f32)
    h = jnp.maximum(h + b1_ref[...], 0.0)
    h = jnp.maximum(
        jnp.dot(h, w2_ref[...], preferred_element_type=f32) + b2_ref[...], 0.0)
    logit = jnp.dot(h, w3_ref[...], preferred_element_type=f32) + b3_ref[...]
    out_ref[...] = jax.nn.sigmoid(logit)


def _full(shape):
    return pl.BlockSpec(shape, lambda i: (0, 0))


_mlp = pl.pallas_call(
    _mlp_body,
    grid=(B // BB,),
    in_specs=[
        pl.BlockSpec((BB, D), lambda i: (i, 0)),
        pl.BlockSpec((BB, D), lambda i: (i, 0)),
        pl.BlockSpec((BB, TAG), lambda i: (i, 0)),
        _full((TAG, D)),
        _full((D, TAG)),
        _full((D, TAG)),
        _full((D, TAG)),
        _full((1, TAG)),
        _full((TAG, D)),
        _full((1, D)),
        _full((D, 1)),
        _full((1, 1)),
    ],
    out_specs=pl.BlockSpec((BB, 1), lambda i: (i, 0)),
    out_shape=jax.ShapeDtypeStruct((B, 1), jnp.float32),
)


def kernel(user_idx, artist_idx, tag_features, user_emb, artist_emb,
           W_tag, W1, b1, W2, b2, W3, b3):
    uidx = user_idx.astype(jnp.int32).reshape(NW, NCH, CH)
    aidx = artist_idx.astype(jnp.int32).reshape(NW, NCH, CH)
    u, a = _gather(uidx, aidx, user_emb, artist_emb)
    out = _mlp(u, a, tag_features,
               W_tag.T,
               W1[:, :D].T, W1[:, D:2 * D].T, W1[:, 2 * D:].T,
               b1.reshape(1, -1),
               W2.T, b2.reshape(1, -1),
               W3.T, b3.reshape(1, 1))
    return out.reshape(B)
